# baseline (speedup 1.0000x reference)
import functools

import jax
import jax.numpy as jnp
from jax import lax
from jax.experimental import pallas as pl
from jax.experimental.pallas import tpu as pltpu
from jax.experimental.pallas import tpu_sc as plsc

N_NODES = 10000
D_FEAT = 128
N_EDGES = 320000

NC = 2
NS = 16
NW = NC * NS

B_W = N_EDGES // NW
CHUNK = 80
N_CHUNK = B_W // CHUNK
GFAN = 2
GROUP = CHUNK * GFAN
N_GROUP = B_W // GROUP
TAIL = B_W - N_GROUP * GROUP
NBUF = 2
STAGE_SPLIT = 4


def _make_gather():
    mesh = plsc.VectorSubcoreMesh(
        core_axis_name="c", subcore_axis_name="s", num_cores=NC, num_subcores=NS
    )

    @functools.partial(
        pl.kernel,
        out_type=jax.ShapeDtypeStruct((N_EDGES, D_FEAT), jnp.float32),
        mesh=mesh,
        scratch_types=[
            pltpu.VMEM((B_W,), jnp.int32),
            pltpu.VMEM((NBUF, GROUP, D_FEAT), jnp.float32),
            pltpu.VMEM_SHARED((N_NODES, D_FEAT), jnp.float32),
            pltpu.SemaphoreType.DMA((NBUF,)),
            pltpu.SemaphoreType.DMA((NBUF,)),
            pltpu.SemaphoreType.DMA,
        ],
    )
    def gather_kernel(table_hbm, idx_hbm, out_hbm, idx_v, rows_v, table_sp,
                      gsem, ssem, tsem):
        sid = lax.axis_index("s")
        wid = sid * NC + lax.axis_index("c")
        base = wid * B_W

        @pl.when(sid < NS - 1)
        def _():
            step = 640 // STAGE_SPLIT
            for j in range(STAGE_SPLIT):
                pltpu.async_copy(
                    table_hbm.at[pl.ds(sid * 640 + j * step, step)],
                    table_sp.at[pl.ds(sid * 640 + j * step, step)],
                    tsem,
                )

        @pl.when(sid == NS - 1)
        def _():
            for off, sz in ((0, 160), (160, 160), (320, 80)):
                pltpu.async_copy(
                    table_hbm.at[pl.ds((NS - 1) * 640 + off, sz)],
                    table_sp.at[pl.ds((NS - 1) * 640 + off, sz)],
                    tsem,
                )

        pltpu.sync_copy(idx_hbm.at[wid], idx_v)

        @pl.when(sid < NS - 1)
        def _():
            pltpu.make_async_copy(
                table_hbm.at[pl.ds(0, 640)],
                table_sp.at[pl.ds(0, 640)],
                tsem,
            ).wait()

        @pl.when(sid == NS - 1)
        def _():
            pltpu.make_async_copy(
                table_hbm.at[pl.ds(0, 400)],
                table_sp.at[pl.ds(0, 400)],
                tsem,
            ).wait()

        plsc.subcore_barrier()

        def start_gathers(g, b):
            for k in range(GFAN):
                pltpu.async_copy(
                    table_sp.at[idx_v.at[pl.ds((g * GFAN + k) * CHUNK, CHUNK)]],
                    rows_v.at[b].at[pl.ds(k * CHUNK, CHUNK)],
                    gsem.at[b],
                )

        def wait_gathers(b):
            pltpu.make_async_copy(
                table_hbm.at[pl.ds(0, GROUP)], rows_v.at[b], gsem.at[b]
            ).wait()

        def store_slot(g):
            return out_hbm.at[pl.ds(base + g * GROUP, GROUP)]

        def start_store(g, b):
            pltpu.async_copy(rows_v.at[b], store_slot(g), ssem.at[b])

        def wait_store(g, b):
            pltpu.make_async_copy(rows_v.at[b], store_slot(g), ssem.at[b]).wait()

        for b in range(NBUF - 1):
            start_gathers(b, b)

        main_end = ((N_GROUP - (NBUF - 1)) // NBUF) * NBUF

        @pl.loop(0, main_end, step=NBUF)
        def _ring(g0):
            for b in range(NBUF):
                g = g0 + b
                b_prev = (b - 1) % NBUF
                wait_gathers(b)
                start_store(g, b)

                @pl.when(g > 0)
                def _():
                    wait_store(g - 1, b_prev)

                @pl.when(g + NBUF - 1 < N_GROUP)
                def _():
                    start_gathers(g + NBUF - 1, b_prev)

        for g in range(main_end, N_GROUP):
            b = g % NBUF
            b_prev = (b - 1) % NBUF
            wait_gathers(b)
            start_store(g, b)
            if g > 0:
                wait_store(g - 1, b_prev)
            if g + NBUF - 1 < N_GROUP:
                start_gathers(g + NBUF - 1, b_prev)

        if TAIL:
            pltpu.async_copy(
                table_sp.at[idx_v.at[pl.ds(N_GROUP * GROUP, TAIL)]],
                rows_v.at[0].at[pl.ds(0, TAIL)],
                gsem.at[0],
            )
            pltpu.make_async_copy(
                table_hbm.at[pl.ds(0, TAIL)],
                rows_v.at[0].at[pl.ds(0, TAIL)],
                gsem.at[0],
            ).wait()
            pltpu.async_copy(
                rows_v.at[0].at[pl.ds(0, TAIL)],
                out_hbm.at[pl.ds(base + N_GROUP * GROUP, TAIL)],
                ssem.at[0],
            )

        wait_store(N_GROUP - 1, (N_GROUP - 1) % NBUF)

        if TAIL:
            pltpu.make_async_copy(
                rows_v.at[0].at[pl.ds(0, TAIL)],
                out_hbm.at[pl.ds(base + N_GROUP * GROUP, TAIL)],
                ssem.at[0],
            ).wait()

    return gather_kernel


_gather = _make_gather()


@jax.jit
def kernel(V_set, node_ids):
    table = V_set[0]
    idx = node_ids.reshape(NW, B_W)
    out = _gather(table, idx)
    return out[jnp.newaxis]


# speedup vs baseline: 1.0515x; 1.0515x over previous
import functools

import jax
import jax.numpy as jnp
from jax import lax
from jax.experimental import pallas as pl
from jax.experimental.pallas import tpu as pltpu
from jax.experimental.pallas import tpu_sc as plsc

N_NODES = 10000
D_FEAT = 128
N_EDGES = 320000

NC = 2
NS = 16
NW = NC * NS

B_W = N_EDGES // NW
CHUNK = 80
N_CHUNK = B_W // CHUNK
GFAN = 1
GROUP = CHUNK * GFAN
N_GROUP = B_W // GROUP
NBUF = 4
STAGE_SPLIT = 8


def _make_gather():
    mesh = plsc.VectorSubcoreMesh(
        core_axis_name="c", subcore_axis_name="s", num_cores=NC, num_subcores=NS
    )

    @functools.partial(
        pl.kernel,
        out_type=jax.ShapeDtypeStruct((N_EDGES, D_FEAT), jnp.float32),
        mesh=mesh,
        scratch_types=[
            pltpu.VMEM((B_W,), jnp.int32),
            pltpu.VMEM((NBUF, GROUP, D_FEAT), jnp.float32),
            pltpu.VMEM_SHARED((N_NODES, D_FEAT), jnp.float32),
            pltpu.SemaphoreType.DMA((NBUF,)),
            pltpu.SemaphoreType.DMA((NBUF,)),
            pltpu.SemaphoreType.DMA,
        ],
    )
    def gather_kernel(table_hbm, idx_hbm, out_hbm, idx_v, rows_v, table_sp,
                      gsem, ssem, tsem):
        sid = lax.axis_index("s")
        wid = sid * NC + lax.axis_index("c")
        base = wid * B_W

        @pl.when(sid < NS - 1)
        def _():
            step = 640 // STAGE_SPLIT
            for j in range(STAGE_SPLIT):
                pltpu.async_copy(
                    table_hbm.at[pl.ds(sid * 640 + j * step, step)],
                    table_sp.at[pl.ds(sid * 640 + j * step, step)],
                    tsem,
                )

        @pl.when(sid == NS - 1)
        def _():
            for off, sz in ((0, 160), (160, 160), (320, 80)):
                pltpu.async_copy(
                    table_hbm.at[pl.ds((NS - 1) * 640 + off, sz)],
                    table_sp.at[pl.ds((NS - 1) * 640 + off, sz)],
                    tsem,
                )

        pltpu.sync_copy(idx_hbm.at[wid], idx_v)

        @pl.when(sid < NS - 1)
        def _():
            pltpu.make_async_copy(
                table_hbm.at[pl.ds(0, 640)],
                table_sp.at[pl.ds(0, 640)],
                tsem,
            ).wait()

        @pl.when(sid == NS - 1)
        def _():
            pltpu.make_async_copy(
                table_hbm.at[pl.ds(0, 400)],
                table_sp.at[pl.ds(0, 400)],
                tsem,
            ).wait()

        plsc.subcore_barrier()

        def start_gathers(g, b):
            for k in range(GFAN):
                pltpu.async_copy(
                    table_sp.at[idx_v.at[pl.ds((g * GFAN + k) * CHUNK, CHUNK)]],
                    rows_v.at[b].at[pl.ds(k * CHUNK, CHUNK)],
                    gsem.at[b],
                )

        def wait_gathers(b):
            pltpu.make_async_copy(
                table_hbm.at[pl.ds(0, GROUP)], rows_v.at[b], gsem.at[b]
            ).wait()

        def store_slot(g):
            return out_hbm.at[pl.ds(base + g * GROUP, GROUP)]

        def start_store(g, b):
            pltpu.async_copy(rows_v.at[b], store_slot(g), ssem.at[b])

        def wait_store(g, b):
            pltpu.make_async_copy(rows_v.at[b], store_slot(g), ssem.at[b]).wait()

        for b in range(NBUF - 1):
            start_gathers(b, b)

        main_end = ((N_GROUP - (NBUF - 1)) // NBUF) * NBUF

        @pl.loop(0, main_end, step=NBUF)
        def _ring(g0):
            for b in range(NBUF):
                g = g0 + b
                b_prev = (b - 1) % NBUF

                @pl.when(g > 0)
                def _():
                    wait_store(g - 1, b_prev)

                @pl.when(g + NBUF - 1 < N_GROUP)
                def _():
                    start_gathers(g + NBUF - 1, b_prev)

                wait_gathers(b)
                start_store(g, b)

        for g in range(main_end, N_GROUP):
            b = g % NBUF
            b_prev = (b - 1) % NBUF
            if g > 0:
                wait_store(g - 1, b_prev)
            if g + NBUF - 1 < N_GROUP:
                start_gathers(g + NBUF - 1, b_prev)
            wait_gathers(b)
            start_store(g, b)

        wait_store(N_GROUP - 1, (N_GROUP - 1) % NBUF)

    return gather_kernel


_gather = _make_gather()


@jax.jit
def kernel(V_set, node_ids):
    table = V_set[0]
    idx = node_ids.reshape(NW, B_W)
    out = _gather(table, idx)
    return out[jnp.newaxis]


# speedup vs baseline: 1.0518x; 1.0003x over previous
import functools

import jax
import jax.numpy as jnp
from jax import lax
from jax.experimental import pallas as pl
from jax.experimental.pallas import tpu as pltpu
from jax.experimental.pallas import tpu_sc as plsc

N_NODES = 10000
D_FEAT = 128
N_EDGES = 320000

NC = 2
NS = 16
NW = NC * NS

B_W = N_EDGES // NW
CHUNK = 80
N_CHUNK = B_W // CHUNK
GFAN = 1
GROUP = CHUNK * GFAN
N_GROUP = B_W // GROUP
NBUF = 4
STAGE_SPLIT = 8


def _make_gather():
    mesh = plsc.VectorSubcoreMesh(
        core_axis_name="c", subcore_axis_name="s", num_cores=NC, num_subcores=NS
    )

    @functools.partial(
        pl.kernel,
        out_type=jax.ShapeDtypeStruct((N_EDGES, D_FEAT), jnp.float32),
        mesh=mesh,
        scratch_types=[
            pltpu.VMEM((B_W,), jnp.int32),
            pltpu.VMEM((NBUF, GROUP, D_FEAT), jnp.float32),
            pltpu.VMEM_SHARED((N_NODES, D_FEAT), jnp.float32),
            pltpu.SemaphoreType.DMA((NBUF,)),
            pltpu.SemaphoreType.DMA((NBUF,)),
            pltpu.SemaphoreType.DMA,
        ],
    )
    def gather_kernel(table_hbm, idx_hbm, out_hbm, idx_v, rows_v, table_sp,
                      gsem, ssem, tsem):
        sid = lax.axis_index("s")
        wid = sid * NC + lax.axis_index("c")
        base = wid * B_W

        @pl.when(sid < NS - 1)
        def _():
            step = 640 // STAGE_SPLIT
            for j in range(STAGE_SPLIT):
                pltpu.async_copy(
                    table_hbm.at[pl.ds(sid * 640 + j * step, step)],
                    table_sp.at[pl.ds(sid * 640 + j * step, step)],
                    tsem,
                )

        @pl.when(sid == NS - 1)
        def _():
            for off, sz in ((0, 160), (160, 160), (320, 80)):
                pltpu.async_copy(
                    table_hbm.at[pl.ds((NS - 1) * 640 + off, sz)],
                    table_sp.at[pl.ds((NS - 1) * 640 + off, sz)],
                    tsem,
                )

        pltpu.sync_copy(idx_hbm.at[wid], idx_v)

        @pl.when(sid < NS - 1)
        def _():
            pltpu.make_async_copy(
                table_hbm.at[pl.ds(0, 640)],
                table_sp.at[pl.ds(0, 640)],
                tsem,
            ).wait()

        @pl.when(sid == NS - 1)
        def _():
            pltpu.make_async_copy(
                table_hbm.at[pl.ds(0, 400)],
                table_sp.at[pl.ds(0, 400)],
                tsem,
            ).wait()

        plsc.subcore_barrier()

        def start_gathers(g, b):
            for k in range(GFAN):
                pltpu.async_copy(
                    table_sp.at[idx_v.at[pl.ds((g * GFAN + k) * CHUNK, CHUNK)]],
                    rows_v.at[b].at[pl.ds(k * CHUNK, CHUNK)],
                    gsem.at[b],
                )

        def wait_gathers(b):
            pltpu.make_async_copy(
                table_hbm.at[pl.ds(0, GROUP)], rows_v.at[b], gsem.at[b]
            ).wait()

        def store_slot(g):
            return out_hbm.at[pl.ds(base + g * GROUP, GROUP)]

        def start_store(g, b):
            pltpu.async_copy(rows_v.at[b], store_slot(g), ssem.at[b])

        def wait_store(g, b):
            pltpu.make_async_copy(rows_v.at[b], store_slot(g), ssem.at[b]).wait()

        for b in range(2):
            start_gathers(b, b)

        main_end = ((N_GROUP - 2) // NBUF) * NBUF

        @pl.loop(0, main_end, step=NBUF)
        def _ring(g0):
            for b in range(NBUF):
                g = g0 + b
                b_next = (b + 2) % NBUF

                @pl.when(g > 1)
                def _():
                    wait_store(g - 2, b_next)

                @pl.when(g + 2 < N_GROUP)
                def _():
                    start_gathers(g + 2, b_next)

                wait_gathers(b)
                start_store(g, b)

        for g in range(main_end, N_GROUP):
            b = g % NBUF
            b_next = (b + 2) % NBUF
            if g > 1:
                wait_store(g - 2, b_next)
            if g + 2 < N_GROUP:
                start_gathers(g + 2, b_next)
            wait_gathers(b)
            start_store(g, b)

        wait_store(N_GROUP - 2, (N_GROUP - 2) % NBUF)
        wait_store(N_GROUP - 1, (N_GROUP - 1) % NBUF)

    return gather_kernel


_gather = _make_gather()


@jax.jit
def kernel(V_set, node_ids):
    table = V_set[0]
    idx = node_ids.reshape(NW, B_W)
    out = _gather(table, idx)
    return out[jnp.newaxis]
